# baseline (speedup 1.0000x reference)
import functools

import jax
import jax.numpy as jnp
from jax import lax
from jax.experimental import pallas as pl
from jax.experimental.pallas import tpu as pltpu
from jax.experimental.pallas import tpu_sc as plsc

_EPS = 1e-6
_TAU = 2.5e-3
_GCAP = 96
_TBLK = 32
_NCANON = _GCAP // _TBLK


def _rot_scalars(pq, s):
    n = jnp.sqrt(jnp.sum(pq * pq, axis=1, keepdims=True))
    u = pq / jnp.maximum(n, _EPS)
    uu = jnp.sum(u * u, axis=1, keepdims=True)
    dot = jnp.sum(u, axis=1, keepdims=True) * s
    return u, uu, dot, 1.0 + dot + _EPS


def _rowdots(u, x, s):
    vx = jnp.sum(x, axis=1, keepdims=True) * s
    ux = jnp.sum(u * x, axis=1, keepdims=True)
    return vx, ux


def _gate_body(x_ref, pq_ref, ct_ref, idx_s, sl_s, gx_s, gpq_s):
    B, D = x_ref.shape
    K = ct_ref.shape[1]
    s = 1.0 / jnp.sqrt(jnp.float32(D))

    x = x_ref[...]
    pq = pq_ref[...]
    ct = ct_ref[...]

    u, uu, dot, denom = _rot_scalars(pq, s)
    vx, ux = _rowdots(u, x, s)
    xc = (x
          + u * (-vx + (dot * vx - ux) / denom)
          + s * (ux - (uu * vx - dot * ux) / denom))

    cn = jnp.sum(ct * ct, axis=0, keepdims=True)
    scores = jnp.dot(xc, ct, preferred_element_type=jnp.float32,
                     precision=lax.Precision.HIGHEST)
    d2 = cn - 2.0 * scores
    m1 = jnp.min(d2, axis=1, keepdims=True)
    kio = lax.broadcasted_iota(jnp.int32, d2.shape, 1)
    idx = jnp.min(jnp.where(d2 == m1, kio, K), axis=1, keepdims=True)
    idx_s[...] = idx

    m2 = jnp.min(jnp.where(kio == idx, jnp.float32(1e30), d2),
                 axis=1, keepdims=True)
    xn = jnp.sum(xc * xc, axis=1, keepdims=True)
    gap = (jnp.sqrt(jnp.maximum(m2 + xn, 0.0))
           - jnp.sqrt(jnp.maximum(m1 + xn, 0.0)))
    flag = gap < _TAU

    ii = lax.broadcasted_iota(jnp.int32, (B, B), 0)
    jj = lax.broadcasted_iota(jnp.int32, (B, B), 1)
    lower = (jj <= ii).astype(jnp.float32)
    fcol = flag.astype(jnp.float32)
    csum = jnp.dot(lower, fcol, preferred_element_type=jnp.float32)
    sl = jnp.where(flag, csum.astype(jnp.int32) - 1, -1)
    sl_s[...] = sl

    slr = sl.reshape(1, B)
    sio = lax.broadcasted_iota(jnp.int32, (_GCAP, B), 0)
    pt = (sio == slr).astype(jnp.float32)
    gx_s[...] = jnp.dot(pt, x, preferred_element_type=jnp.float32,
                        precision=lax.Precision.HIGHEST)
    gpq_s[...] = jnp.dot(pt, pq, preferred_element_type=jnp.float32,
                         precision=lax.Precision.HIGHEST)


def _canon_body(base, gx_s, gpq_s, gxc_s):
    T = _TBLK
    D = gx_s.shape[1]
    s = 1.0 / jnp.sqrt(jnp.float32(D))

    x = gx_s[pl.ds(base, T), :]
    pq = gpq_s[pl.ds(base, T), :]
    n = jnp.sqrt(jnp.sum(pq * pq, axis=1, keepdims=True))
    u = pq / jnp.maximum(n, _EPS)
    p = u * s
    pT = p.T
    dots = jnp.sum(p, axis=1, keepdims=True)

    ii = lax.broadcasted_iota(jnp.int32, (D, D), 0)
    jj = lax.broadcasted_iota(jnp.int32, (D, D), 1)
    eye = (ii == jj).astype(jnp.float32)

    rows = []
    for t in range(T):
        A = pT[:, t:t + 1] - p[t:t + 1, :]
        A2 = jnp.dot(A, A, preferred_element_type=jnp.float32)
        R = eye + A + A2 / (1.0 + dots[t, 0] + _EPS)
        rows.append(jnp.dot(x[t:t + 1, :], R,
                            preferred_element_type=jnp.float32))
    gxc_s[pl.ds(base, T), :] = jnp.concatenate(rows, axis=0)


def _idx_kernel(x_ref, pq_ref, ct_ref, idxo_ref, idx_s, sl_s, gx_s, gpq_s,
                gxc_s):
    pid = pl.program_id(0)

    @pl.when(pid == 0)
    def _():
        _gate_body(x_ref, pq_ref, ct_ref, idx_s, sl_s, gx_s, gpq_s)

    @pl.when((pid >= 1) & (pid <= _NCANON))
    def _():
        _canon_body((pid - 1) * _TBLK, gx_s, gpq_s, gxc_s)

    @pl.when(pid == _NCANON + 1)
    def _():
        B = x_ref.shape[0]
        K = ct_ref.shape[1]
        ct = ct_ref[...]
        sl = sl_s[...]
        gxc = gxc_s[...]
        cn = jnp.sum(ct * ct, axis=0, keepdims=True)
        gsc = jnp.dot(gxc, ct, preferred_element_type=jnp.float32,
                      precision=lax.Precision.HIGHEST)
        gd2 = cn - 2.0 * gsc
        gm = jnp.min(gd2, axis=1, keepdims=True)
        gkio = lax.broadcasted_iota(jnp.int32, gd2.shape, 1)
        gidx = jnp.min(jnp.where(gd2 == gm, gkio, K), axis=1, keepdims=True)
        sio = lax.broadcasted_iota(jnp.int32, (B, _GCAP), 1)
        g = (sio == sl).astype(jnp.float32)
        rep = jnp.dot(g, gidx.astype(jnp.float32),
                      preferred_element_type=jnp.float32,
                      precision=lax.Precision.HIGHEST)
        idxo_ref[...] = jnp.where((sl >= 0) & (sl < _GCAP),
                                  rep.astype(jnp.int32), idx_s[...])


def _rot_kernel(x_ref, pq_ref, qc_ref, q_ref, loss_ref):
    B, D = x_ref.shape
    s = 1.0 / jnp.sqrt(jnp.float32(D))
    x = x_ref[...]
    pq = pq_ref[...]
    qc = qc_ref[...]
    u, uu, dot, denom = _rot_scalars(pq, s)
    vq, uq = _rowdots(u, qc, s)
    quant = (qc
             + u * (vq + (dot * vq - uq) / denom)
             + s * (-uq - (uu * vq - dot * uq) / denom))
    q_ref[...] = quant
    diff = x - quant
    lc = jnp.sum(diff * diff) / jnp.float32(B)
    loss_ref[...] = jnp.reshape(lc + 0.25 * lc, (1, 1))


def _make_sc_gather(K, D, B):
    info = plsc.get_sparse_core_info()
    nw = info.num_cores * info.num_subcores
    b_per_w = B // nw
    mesh = plsc.VectorSubcoreMesh(core_axis_name="c", subcore_axis_name="s")

    @functools.partial(
        pl.kernel, mesh=mesh,
        out_type=jax.ShapeDtypeStruct((B, D), jnp.float32),
        scratch_types=[
            pltpu.VMEM((b_per_w,), jnp.int32),
            pltpu.VMEM((b_per_w, D), jnp.float32),
            pltpu.SemaphoreType.DMA,
        ],
    )
    def sc_gather(table_hbm, idx_hbm, out_hbm, idx_v, rows_v, sem):
        wid = lax.axis_index("s") * info.num_cores + lax.axis_index("c")
        base = wid * b_per_w
        pltpu.sync_copy(idx_hbm.at[pl.ds(base, b_per_w)], idx_v)
        pltpu.async_copy(table_hbm.at[idx_v], rows_v, sem).wait()
        pltpu.sync_copy(rows_v, out_hbm.at[pl.ds(base, b_per_w)])

    return sc_gather


def kernel(x, prev_q, codes):
    B, D = x.shape
    K = codes.shape[1]
    c2d = codes.reshape(K, D)
    ct = c2d.T

    full = lambda shape: pl.BlockSpec(shape, lambda i: tuple(0 for _ in shape))
    idx = pl.pallas_call(
        _idx_kernel,
        grid=(_NCANON + 2,),
        in_specs=[full((B, D)), full((B, D)), full((D, K))],
        out_specs=full((B, 1)),
        out_shape=jax.ShapeDtypeStruct((B, 1), jnp.int32),
        scratch_shapes=[
            pltpu.VMEM((B, 1), jnp.int32),
            pltpu.VMEM((B, 1), jnp.int32),
            pltpu.VMEM((_GCAP, D), jnp.float32),
            pltpu.VMEM((_GCAP, D), jnp.float32),
            pltpu.VMEM((_GCAP, D), jnp.float32),
        ],
    )(x, prev_q, ct)

    qc = _make_sc_gather(K, D, B)(c2d, idx.reshape(B))

    q, loss = pl.pallas_call(
        _rot_kernel,
        out_shape=(
            jax.ShapeDtypeStruct((B, D), jnp.float32),
            jax.ShapeDtypeStruct((1, 1), jnp.float32),
        ),
    )(x, prev_q, qc)
    return q, idx.reshape(B), loss.reshape(())


# speedup vs baseline: 1.5475x; 1.5475x over previous
import functools

import jax
import jax.numpy as jnp
from jax import lax
from jax.experimental import pallas as pl
from jax.experimental.pallas import tpu as pltpu

_EPS = 1e-6
_TAU = 2.5e-3
_GCAP = 96
_TBLK = 32
_NCANON = _GCAP // _TBLK


def _rot_scalars(pq, s):
    n = jnp.sqrt(jnp.sum(pq * pq, axis=1, keepdims=True))
    u = pq / jnp.maximum(n, _EPS)
    uu = jnp.sum(u * u, axis=1, keepdims=True)
    dot = jnp.sum(u, axis=1, keepdims=True) * s
    return u, uu, dot, 1.0 + dot + _EPS


def _rowdots(u, x, s):
    vx = jnp.sum(x, axis=1, keepdims=True) * s
    ux = jnp.sum(u * x, axis=1, keepdims=True)
    return vx, ux


def _gate_body(x_ref, pq_ref, c_ref, idx_s, sl_s, gx_s, gpq_s):
    B, D = x_ref.shape
    K = c_ref.shape[0]
    s = 1.0 / jnp.sqrt(jnp.float32(D))

    x = x_ref[...]
    pq = pq_ref[...]
    c = c_ref[...]

    u, uu, dot, denom = _rot_scalars(pq, s)
    vx, ux = _rowdots(u, x, s)
    xc = (x
          + u * (-vx + (dot * vx - ux) / denom)
          + s * (ux - (uu * vx - dot * ux) / denom))

    cn = jnp.sum(c * c, axis=1, keepdims=True).T
    scores = lax.dot_general(xc, c, (((1,), (1,)), ((), ())),
                             preferred_element_type=jnp.float32,
                             precision=lax.Precision.HIGHEST)
    d2 = cn - 2.0 * scores
    m1 = jnp.min(d2, axis=1, keepdims=True)
    kio = lax.broadcasted_iota(jnp.int32, d2.shape, 1)
    idx = jnp.min(jnp.where(d2 == m1, kio, K), axis=1, keepdims=True)
    idx_s[...] = idx

    m2 = jnp.min(jnp.where(kio == idx, jnp.float32(1e30), d2),
                 axis=1, keepdims=True)
    xn = jnp.sum(xc * xc, axis=1, keepdims=True)
    gap = (jnp.sqrt(jnp.maximum(m2 + xn, 0.0))
           - jnp.sqrt(jnp.maximum(m1 + xn, 0.0)))
    flag = gap < _TAU

    ii = lax.broadcasted_iota(jnp.int32, (B, B), 0)
    jj = lax.broadcasted_iota(jnp.int32, (B, B), 1)
    lower = (jj <= ii).astype(jnp.float32)
    fcol = flag.astype(jnp.float32)
    csum = jnp.dot(lower, fcol, preferred_element_type=jnp.float32)
    sl = jnp.where(flag, csum.astype(jnp.int32) - 1, -1)
    sl_s[...] = sl

    slr = sl.reshape(1, B)
    sio = lax.broadcasted_iota(jnp.int32, (_GCAP, B), 0)
    pt = (sio == slr).astype(jnp.float32)
    gx_s[...] = jnp.dot(pt, x, preferred_element_type=jnp.float32,
                        precision=lax.Precision.HIGHEST)
    gpq_s[...] = jnp.dot(pt, pq, preferred_element_type=jnp.float32,
                         precision=lax.Precision.HIGHEST)


def _canon_body(base, gx_s, gpq_s, gxc_s):
    T = _TBLK
    D = gx_s.shape[1]
    s = 1.0 / jnp.sqrt(jnp.float32(D))

    x = gx_s[pl.ds(base, T), :]
    pq = gpq_s[pl.ds(base, T), :]
    n = jnp.sqrt(jnp.sum(pq * pq, axis=1, keepdims=True))
    u = pq / jnp.maximum(n, _EPS)
    p = u * s
    pT = p.T
    dots = jnp.sum(p, axis=1, keepdims=True)

    ii = lax.broadcasted_iota(jnp.int32, (D, D), 0)
    jj = lax.broadcasted_iota(jnp.int32, (D, D), 1)
    eye = (ii == jj).astype(jnp.float32)

    rows = []
    for t in range(T):
        A = pT[:, t:t + 1] - p[t:t + 1, :]
        A2 = jnp.dot(A, A, preferred_element_type=jnp.float32)
        R = eye + A + A2 / (1.0 + dots[t, 0] + _EPS)
        rows.append(jnp.dot(x[t:t + 1, :], R,
                            preferred_element_type=jnp.float32))
    gxc_s[pl.ds(base, T), :] = jnp.concatenate(rows, axis=0)


def _final_body(x_ref, pq_ref, c_ref, idx_s, sl_s, gxc_s,
                q_ref, idxo_ref, loss_ref):
    B, D = x_ref.shape
    K = c_ref.shape[0]
    s = 1.0 / jnp.sqrt(jnp.float32(D))

    x = x_ref[...]
    pq = pq_ref[...]
    c = c_ref[...]
    sl = sl_s[...]
    gxc = gxc_s[...]

    cn = jnp.sum(c * c, axis=1, keepdims=True).T
    gsc = lax.dot_general(gxc, c, (((1,), (1,)), ((), ())),
                          preferred_element_type=jnp.float32,
                          precision=lax.Precision.HIGHEST)
    gd2 = cn - 2.0 * gsc
    gm = jnp.min(gd2, axis=1, keepdims=True)
    gkio = lax.broadcasted_iota(jnp.int32, gd2.shape, 1)
    gidx = jnp.min(jnp.where(gd2 == gm, gkio, K), axis=1, keepdims=True)

    sio = lax.broadcasted_iota(jnp.int32, (B, _GCAP), 1)
    g = (sio == sl).astype(jnp.float32)
    rep = jnp.dot(g, gidx.astype(jnp.float32),
                  preferred_element_type=jnp.float32,
                  precision=lax.Precision.HIGHEST)
    idx = jnp.where((sl >= 0) & (sl < _GCAP),
                    rep.astype(jnp.int32), idx_s[...])
    idxo_ref[...] = idx

    kio = lax.broadcasted_iota(jnp.int32, (B, K), 1)
    oh = (kio == idx).astype(jnp.float32)
    qc = jnp.dot(oh, c_ref[...], preferred_element_type=jnp.float32,
                 precision=lax.Precision.HIGHEST)
    u, uu, dot, denom = _rot_scalars(pq, s)
    vq, uq = _rowdots(u, qc, s)
    quant = (qc
             + u * (vq + (dot * vq - uq) / denom)
             + s * (-uq - (uu * vq - dot * uq) / denom))
    q_ref[...] = quant

    diff = x - quant
    lc = jnp.sum(diff * diff) / jnp.float32(B)
    loss_ref[...] = jnp.reshape(lc + 0.25 * lc, (1, 1))


def _vq_kernel(x_ref, pq_ref, c_ref, q_ref, idxo_ref, loss_ref,
               idx_s, sl_s, gx_s, gpq_s, gxc_s):
    pid = pl.program_id(0)

    @pl.when(pid == 0)
    def _():
        _gate_body(x_ref, pq_ref, c_ref, idx_s, sl_s, gx_s, gpq_s)

    @pl.when((pid >= 1) & (pid <= _NCANON))
    def _():
        _canon_body((pid - 1) * _TBLK, gx_s, gpq_s, gxc_s)

    @pl.when(pid == _NCANON + 1)
    def _():
        _final_body(x_ref, pq_ref, c_ref, idx_s, sl_s, gxc_s,
                    q_ref, idxo_ref, loss_ref)


def kernel(x, prev_q, codes):
    B, D = x.shape
    K = codes.shape[1]
    c2d = codes.reshape(K, D)

    full = lambda shape: pl.BlockSpec(shape, lambda i: tuple(0 for _ in shape))
    q, idx, loss = pl.pallas_call(
        _vq_kernel,
        grid=(_NCANON + 2,),
        in_specs=[full((B, D)), full((B, D)), full((K, D))],
        out_specs=(full((B, D)), full((B, 1)), full((1, 1))),
        out_shape=(
            jax.ShapeDtypeStruct((B, D), jnp.float32),
            jax.ShapeDtypeStruct((B, 1), jnp.int32),
            jax.ShapeDtypeStruct((1, 1), jnp.float32),
        ),
        scratch_shapes=[
            pltpu.VMEM((B, 1), jnp.int32),
            pltpu.VMEM((B, 1), jnp.int32),
            pltpu.VMEM((_GCAP, D), jnp.float32),
            pltpu.VMEM((_GCAP, D), jnp.float32),
            pltpu.VMEM((_GCAP, D), jnp.float32),
        ],
    )(x, prev_q, c2d)
    return q, idx.reshape(B), loss.reshape(())


# speedup vs baseline: 1.5512x; 1.0023x over previous
import jax
import jax.numpy as jnp
from jax import lax
from jax.experimental import pallas as pl
from jax.experimental.pallas import tpu as pltpu

_EPS = 1e-6
_TAU = 2.5e-3
_GCAP = 96
_TBLK = 32
_NCANON = _GCAP // _TBLK


def _rot_scalars(pq, s):
    n = jnp.sqrt(jnp.sum(pq * pq, axis=1, keepdims=True))
    u = pq / jnp.maximum(n, _EPS)
    uu = jnp.sum(u * u, axis=1, keepdims=True)
    dot = jnp.sum(u, axis=1, keepdims=True) * s
    return u, uu, dot, 1.0 + dot + _EPS


def _rowdots(u, x, s):
    vx = jnp.sum(x, axis=1, keepdims=True) * s
    ux = jnp.sum(u * x, axis=1, keepdims=True)
    return vx, ux


def _gate_body(x_ref, pq_ref, c_ref, idx_s, sl_s, gx_s, gpq_s):
    B, D = x_ref.shape
    K = c_ref.shape[0]
    s = 1.0 / jnp.sqrt(jnp.float32(D))

    x = x_ref[...]
    pq = pq_ref[...]
    c = c_ref[...]

    u, uu, dot, denom = _rot_scalars(pq, s)
    vx, ux = _rowdots(u, x, s)
    xc = (x
          + u * (-vx + (dot * vx - ux) / denom)
          + s * (ux - (uu * vx - dot * ux) / denom))

    cn = jnp.sum(c * c, axis=1, keepdims=True).T
    scores = lax.dot_general(xc, c, (((1,), (1,)), ((), ())),
                             preferred_element_type=jnp.float32,
                             precision=lax.Precision.HIGHEST)
    d2 = cn - 2.0 * scores
    m1 = jnp.min(d2, axis=1, keepdims=True)
    kio = lax.broadcasted_iota(jnp.int32, d2.shape, 1)
    idx = jnp.min(jnp.where(d2 == m1, kio, K), axis=1, keepdims=True)
    idx_s[...] = idx

    m2 = jnp.min(jnp.where(kio == idx, jnp.float32(1e30), d2),
                 axis=1, keepdims=True)
    xn = jnp.sum(xc * xc, axis=1, keepdims=True)
    gap = (jnp.sqrt(jnp.maximum(m2 + xn, 0.0))
           - jnp.sqrt(jnp.maximum(m1 + xn, 0.0)))
    flag = gap < _TAU

    ii = lax.broadcasted_iota(jnp.int32, (B, B), 0)
    jj = lax.broadcasted_iota(jnp.int32, (B, B), 1)
    lower = (jj <= ii).astype(jnp.float32)
    fcol = flag.astype(jnp.float32)
    csum = jnp.dot(lower, fcol, preferred_element_type=jnp.float32)
    sl = jnp.where(flag, csum.astype(jnp.int32) - 1, -1)
    sl_s[...] = sl

    slr = sl.reshape(1, B)
    sio = lax.broadcasted_iota(jnp.int32, (_GCAP, B), 0)
    pt = (sio == slr).astype(jnp.float32)
    gx_s[...] = jnp.dot(pt, x, preferred_element_type=jnp.float32,
                        precision=lax.Precision.HIGHEST)
    gpq_s[...] = jnp.dot(pt, pq, preferred_element_type=jnp.float32,
                         precision=lax.Precision.HIGHEST)


def _canon_body(base, gx_s, gpq_s, gxc_s):
    T = _TBLK
    D = gx_s.shape[1]
    s = 1.0 / jnp.sqrt(jnp.float32(D))

    x = gx_s[pl.ds(base, T), :]
    pq = gpq_s[pl.ds(base, T), :]
    n = jnp.sqrt(jnp.sum(pq * pq, axis=1, keepdims=True))
    u = pq / jnp.maximum(n, _EPS)
    p = u * s
    pT = p.T
    dots = jnp.sum(p, axis=1, keepdims=True)

    ii = lax.broadcasted_iota(jnp.int32, (D, D), 0)
    jj = lax.broadcasted_iota(jnp.int32, (D, D), 1)
    eye = (ii == jj).astype(jnp.float32)

    rows = []
    for t in range(T):
        A = pT[:, t:t + 1] - p[t:t + 1, :]
        A2 = jnp.dot(A, A, preferred_element_type=jnp.float32)
        R = eye + A + A2 / (1.0 + dots[t, 0] + _EPS)
        rows.append(jnp.dot(x[t:t + 1, :], R,
                            preferred_element_type=jnp.float32))
    gxc_s[pl.ds(base, T), :] = jnp.concatenate(rows, axis=0)


def _final_body(x_ref, pq_ref, c_ref, idx_s, sl_s, gxc_s,
                q_ref, idxo_ref, loss_ref):
    B, D = x_ref.shape
    K = c_ref.shape[0]
    s = 1.0 / jnp.sqrt(jnp.float32(D))

    x = x_ref[...]
    pq = pq_ref[...]
    c = c_ref[...]
    sl = sl_s[...]
    gxc = gxc_s[...]

    cn = jnp.sum(c * c, axis=1, keepdims=True).T
    gsc = lax.dot_general(gxc, c, (((1,), (1,)), ((), ())),
                          preferred_element_type=jnp.float32,
                          precision=lax.Precision.HIGHEST)
    gd2 = cn - 2.0 * gsc
    gm = jnp.min(gd2, axis=1, keepdims=True)
    gkio = lax.broadcasted_iota(jnp.int32, gd2.shape, 1)
    gidx = jnp.min(jnp.where(gd2 == gm, gkio, K), axis=1, keepdims=True)

    sio = lax.broadcasted_iota(jnp.int32, (B, _GCAP), 1)
    g = (sio == sl).astype(jnp.float32)
    rep = jnp.dot(g, gidx.astype(jnp.float32),
                  preferred_element_type=jnp.float32,
                  precision=lax.Precision.HIGHEST)
    idx = jnp.where((sl >= 0) & (sl < _GCAP),
                    rep.astype(jnp.int32), idx_s[...])
    idxo_ref[...] = idx

    kio = lax.broadcasted_iota(jnp.int32, (B, K), 1)
    oh = (kio == idx).astype(jnp.float32)
    qc = jnp.dot(oh, c_ref[...], preferred_element_type=jnp.float32,
                 precision=lax.Precision.HIGHEST)
    u, uu, dot, denom = _rot_scalars(pq, s)
    vq, uq = _rowdots(u, qc, s)
    quant = (qc
             + u * (vq + (dot * vq - uq) / denom)
             + s * (-uq - (uu * vq - dot * uq) / denom))
    q_ref[...] = quant

    diff = x - quant
    lc = jnp.sum(diff * diff) / jnp.float32(B)
    loss_ref[...] = jnp.reshape(lc + 0.25 * lc, (1, 1))


def _vq_kernel(x_ref, pq_ref, c_ref, q_ref, idxo_ref, loss_ref,
               idx_s, sl_s, gx_s, gpq_s, gxc_s):
    pid = pl.program_id(0)

    @pl.when(pid == 0)
    def _():
        _gate_body(x_ref, pq_ref, c_ref, idx_s, sl_s, gx_s, gpq_s)

    @pl.when((pid >= 1) & (pid <= _NCANON))
    def _():
        _canon_body((pid - 1) * _TBLK, gx_s, gpq_s, gxc_s)

    @pl.when(pid == _NCANON + 1)
    def _():
        _final_body(x_ref, pq_ref, c_ref, idx_s, sl_s, gxc_s,
                    q_ref, idxo_ref, loss_ref)


def kernel(x, prev_q, codes):
    B, D = x.shape
    K = codes.shape[1]
    c2d = codes.reshape(K, D)

    full = lambda shape: pl.BlockSpec(shape, lambda i: tuple(0 for _ in shape))
    q, idx, loss = pl.pallas_call(
        _vq_kernel,
        grid=(_NCANON + 2,),
        in_specs=[full((B, D)), full((B, D)), full((K, D))],
        out_specs=(full((B, D)), full((B, 1)), full((1, 1))),
        out_shape=(
            jax.ShapeDtypeStruct((B, D), jnp.float32),
            jax.ShapeDtypeStruct((B, 1), jnp.int32),
            jax.ShapeDtypeStruct((1, 1), jnp.float32),
        ),
        scratch_shapes=[
            pltpu.VMEM((B, 1), jnp.int32),
            pltpu.VMEM((B, 1), jnp.int32),
            pltpu.VMEM((_GCAP, D), jnp.float32),
            pltpu.VMEM((_GCAP, D), jnp.float32),
            pltpu.VMEM((_GCAP, D), jnp.float32),
        ],
    )(x, prev_q, c2d)
    return q, idx.reshape(B), loss.reshape(())
